# baseline (speedup 1.0000x reference)
import functools

import jax
import jax.numpy as jnp
from jax import lax
from jax.experimental import pallas as pl
from jax.experimental.pallas import tpu as pltpu
from jax.experimental.pallas import tpu_sc as plsc

MU_C = 0.1
NC = 2
NS = 16
CHUNK = 128


def _fill_f32(ref, rows, width, value):
    vals = jnp.full((16,), value, jnp.float32)

    def body(i, carry):
        for t in range(width // 16):
            ref[i, pl.ds(t * 16, 16)] = vals
        return carry

    lax.fori_loop(0, rows, body, 0)


def _unpack_idx(src_packed, dst32):
    for t in range(CHUNK // 32):
        v = src_packed[pl.ds(t * 16, 16)]
        dst32[pl.ds(t * 32, 16)] = jnp.bitwise_and(v, 0xFFFF)
        dst32[pl.ds(t * 32 + 16, 16)] = lax.shift_right_logical(v, 16)


def _make_deg_pass(npad, nchunks):
    w = 16
    mesh = plsc.VectorSubcoreMesh(core_axis_name="c", subcore_axis_name="s")
    cpt = nchunks // (NC * NS)
    rps = npad // NS
    ncopy = rps // CHUNK

    scratch = [
        pltpu.VMEM((CHUNK, w), jnp.float32),
        pltpu.VMEM((CHUNK // 2,), jnp.int32),
        pltpu.VMEM((CHUNK,), jnp.int32),
        pltpu.VMEM_SHARED((npad, w), jnp.float32),
        pltpu.SemaphoreType.DMA,
    ]
    out_type = jax.ShapeDtypeStruct((NC, npad, w), jnp.float32)

    @functools.partial(
        pl.kernel, out_type=out_type, mesh=mesh, scratch_types=scratch,
        compiler_params=pltpu.CompilerParams(use_tc_tiling_on_sc=False))
    def deg_pass(row_hbm, out_hbm, ones_v, rp_v, ridx_v, acc, sem):
        c = lax.axis_index("c")
        s = lax.axis_index("s")
        tid = c * NS + s

        _fill_f32(ones_v, CHUNK, w, 0.0)
        for k in range(ncopy):
            pltpu.sync_copy(ones_v,
                            acc.at[pl.ds(s * rps + k * CHUNK, CHUNK)])
        _fill_f32(ones_v, CHUNK, w, 1.0)
        plsc.subcore_barrier()

        def body(j, carry):
            ch = tid * cpt + j
            pltpu.sync_copy(row_hbm.at[ch], rp_v)
            _unpack_idx(rp_v, ridx_v)
            pltpu.sync_copy(ones_v, acc.at[ridx_v], add=True)
            return carry

        lax.fori_loop(0, cpt, body, 0)
        plsc.subcore_barrier()
        pltpu.sync_copy(acc.at[pl.ds(s * rps, rps)],
                        out_hbm.at[c, pl.ds(s * rps, rps)])

    return deg_pass


def _make_spmm_split(npad, w2, nchunks, deep=True):
    mesh = plsc.VectorSubcoreMesh(core_axis_name="c", subcore_axis_name="s")
    cps = nchunks // NS
    rps = npad // NS
    ncopy = rps // CHUNK

    nset = 4 if deep else 2

    scratch = [
        [pltpu.VMEM((CHUNK, w2), jnp.float32) for _ in range(nset)],
        [pltpu.VMEM((CHUNK,), jnp.int32) for _ in range(nset)],
        [pltpu.VMEM((CHUNK,), jnp.int32) for _ in range(nset)],
        [pltpu.VMEM((CHUNK // 2,), jnp.int32) for _ in range(nset)],
        [pltpu.VMEM((CHUNK // 2,), jnp.int32) for _ in range(nset)],
        pltpu.VMEM_SHARED((npad, w2), jnp.float32),
        pltpu.VMEM_SHARED((npad, w2), jnp.float32),
        [pltpu.SemaphoreType.DMA for _ in range(nset)],
        [pltpu.SemaphoreType.DMA for _ in range(nset)],
        [pltpu.SemaphoreType.DMA for _ in range(nset)],
    ]
    out_type = jax.ShapeDtypeStruct((NC, npad, w2), jnp.float32)

    @functools.partial(
        pl.kernel, out_type=out_type, mesh=mesh, scratch_types=scratch,
        compiler_params=pltpu.CompilerParams(use_tc_tiling_on_sc=False))
    def spmm(t0_hbm, t1_hbm, col_hbm, row_hbm, out_hbm, bufs, rvs, cvs,
             rps_, cps_, tbl, acc, gsems, isems, ssems):
        c = lax.axis_index("c")
        s = lax.axis_index("s")

        for cc in range(NC):
            src = t0_hbm if cc == 0 else t1_hbm
            for ss in range(NS):
                @pl.when(jnp.logical_and(c == cc, s == ss))
                def _():
                    pltpu.sync_copy(src.at[pl.ds(ss * rps, rps)],
                                    tbl.at[pl.ds(ss * rps, rps)])

        _fill_f32(bufs[0], CHUNK, w2, 0.0)
        for k in range(ncopy):
            pltpu.sync_copy(bufs[0],
                            acc.at[pl.ds(s * rps + k * CHUNK, CHUNK)])
        plsc.subcore_barrier()

        base0 = s * cps

        def fetch_idx(p, ch):
            pltpu.async_copy(col_hbm.at[ch], cps_[p], isems[p])
            pltpu.async_copy(row_hbm.at[ch], rps_[p], isems[p])

        def wait_idx(p, ch):
            pltpu.make_async_copy(col_hbm.at[ch], cps_[p], isems[p]).wait()
            pltpu.make_async_copy(row_hbm.at[ch], rps_[p], isems[p]).wait()
            _unpack_idx(cps_[p], cvs[p])
            _unpack_idx(rps_[p], rvs[p])

        def wait_scat(p):
            pltpu.make_async_copy(bufs[p], acc.at[rvs[p]], ssems[p]).wait()

        if not deep:
            fetch_idx(0, base0)
            fetch_idx(1, base0 + 1)

            def body2(g, carry):
                ch = base0 + 2 * g
                wait_idx(0, ch)
                wait_idx(1, ch + 1)
                g0 = pltpu.async_copy(tbl.at[cvs[0]], bufs[0], gsems[0])
                g1 = pltpu.async_copy(tbl.at[cvs[1]], bufs[1], gsems[1])
                g0.wait()
                pltpu.sync_copy(bufs[0], acc.at[rvs[0]], add=True)
                fetch_idx(0, ch + 2)
                g1.wait()
                pltpu.sync_copy(bufs[1], acc.at[rvs[1]], add=True)
                fetch_idx(1, ch + 3)
                return carry

            lax.fori_loop(0, cps // 2, body2, 0)
            wait_idx(0, base0)
            wait_idx(1, base0)
            plsc.subcore_barrier()
            pltpu.sync_copy(acc.at[pl.ds(s * rps, rps)],
                            out_hbm.at[c, pl.ds(s * rps, rps)])
            return

        fetch_idx(0, base0)
        fetch_idx(1, base0 + 1)
        wait_idx(0, base0)
        pltpu.async_copy(tbl.at[cvs[0]], bufs[0], gsems[0])
        for p in (2, 3):
            pltpu.async_copy(bufs[p], acc.at[pl.ds(npad - CHUNK, CHUNK)],
                             ssems[p])

        def stage(p, ch):
            q = (p + 1) % nset
            r = (p + 2) % nset
            pltpu.make_async_copy(tbl.at[cvs[p]], bufs[p], gsems[p]).wait()
            pltpu.async_copy(bufs[p], acc.at[rvs[p]], ssems[p], add=True)
            wait_idx(q, ch + 1)
            pltpu.async_copy(tbl.at[cvs[q]], bufs[q], gsems[q])
            wait_scat(r)
            fetch_idx(r, ch + 2)

        def body(g, carry):
            base = base0 + g * nset
            for p in range(nset):
                stage(p, base + p)
            return carry

        lax.fori_loop(0, cps // nset, body, 0)

        pltpu.make_async_copy(tbl.at[cvs[0]], bufs[0], gsems[0]).wait()
        wait_idx(1, base0)
        wait_scat((cps - 2) % nset)
        wait_scat((cps - 1) % nset)
        plsc.subcore_barrier()
        pltpu.sync_copy(acc.at[pl.ds(s * rps, rps)],
                        out_hbm.at[c, pl.ds(s * rps, rps)])

    return spmm


def _tc_prep(x, w1, b1r, degp, npad, blk):
    n, din = x.shape
    dh = w1.shape[0]
    wt = dh + 32
    w2 = wt // 2
    g = n // blk

    def body(x_ref, w1_ref, b1_ref, degp_ref, h_ref, ta_ref, tb_ref,
             dinv_ref):
        xx = x_ref[...]
        h = lax.dot_general(xx, w1_ref[...], (((1,), (1,)), ((), ())),
                            precision=lax.Precision.HIGHEST)
        h = jnp.maximum(h + b1_ref[...], 0.0)
        deg = degp_ref[0, :, 0] + degp_ref[1, :, 0] + 1.0
        dinv = lax.rsqrt(deg)[:, None]
        h_ref[...] = h
        dinv_ref[...] = dinv
        t1 = jnp.concatenate(
            [dinv * h, dinv, jnp.zeros((blk, wt - dh - 1), jnp.float32)],
            axis=1)
        ta_ref[...] = t1[:, :w2]
        tb_ref[...] = t1[:, w2:]

    return pl.pallas_call(
        body,
        grid=(g,),
        in_specs=[
            pl.BlockSpec((blk, din), lambda i: (i, 0)),
            pl.BlockSpec((dh, din), lambda i: (0, 0)),
            pl.BlockSpec((1, dh), lambda i: (0, 0)),
            pl.BlockSpec((2, blk, 16), lambda i: (0, i, 0)),
        ],
        out_specs=[
            pl.BlockSpec((blk, dh), lambda i: (i, 0)),
            pl.BlockSpec((blk, w2), lambda i: (i, 0)),
            pl.BlockSpec((blk, w2), lambda i: (i, 0)),
            pl.BlockSpec((blk, 1), lambda i: (i, 0)),
        ],
        out_shape=[
            jax.ShapeDtypeStruct((n, dh), jnp.float32),
            jax.ShapeDtypeStruct((npad, w2), jnp.float32),
            jax.ShapeDtypeStruct((npad, w2), jnp.float32),
            jax.ShapeDtypeStruct((n, 1), jnp.float32),
        ],
    )(x, w1, b1r, degp)


def _tc_combine1(parts, dinv, h, npad, blk):
    n, dh = h.shape
    w2 = parts.shape[2]
    g = n // blk

    def body(p_ref, dinv_ref, h_ref, out1_ref, ta_ref, tb_ref, ab_ref):
        p = jnp.concatenate([p_ref[0], p_ref[1]], axis=1)
        s128 = p[:, :dh]
        scol = p[:, dh:dh + 1]
        dv = dinv_ref[...]
        deg_m = dv * scol + dv * dv
        alpha = 1.0 / (MU_C + deg_m)
        beta = MU_C * alpha
        hh = h_ref[...]
        out1 = alpha * (dv * s128 + dv * dv * hh) + beta * hh
        out1_ref[...] = out1
        t2 = dv * out1
        ta_ref[...] = t2[:, :dh // 2]
        tb_ref[...] = t2[:, dh // 2:]
        ab_ref[...] = jnp.concatenate([alpha, beta], axis=1)

    return pl.pallas_call(
        body,
        grid=(g,),
        in_specs=[
            pl.BlockSpec((2, blk, w2), lambda i: (0, i, 0)),
            pl.BlockSpec((blk, 1), lambda i: (i, 0)),
            pl.BlockSpec((blk, dh), lambda i: (i, 0)),
        ],
        out_specs=[
            pl.BlockSpec((blk, dh), lambda i: (i, 0)),
            pl.BlockSpec((blk, dh // 2), lambda i: (i, 0)),
            pl.BlockSpec((blk, dh // 2), lambda i: (i, 0)),
            pl.BlockSpec((blk, 2), lambda i: (i, 0)),
        ],
        out_shape=[
            jax.ShapeDtypeStruct((n, dh), jnp.float32),
            jax.ShapeDtypeStruct((npad, dh // 2), jnp.float32),
            jax.ShapeDtypeStruct((npad, dh // 2), jnp.float32),
            jax.ShapeDtypeStruct((n, 2), jnp.float32),
        ],
    )(parts, dinv, h)


def _tc_combine2(parts, dinv, out1, h, ab, w2_, b2r, blk):
    n, dh = h.shape
    dout = w2_.shape[0]
    w2 = parts.shape[2]
    g = n // blk

    def body(p_ref, dinv_ref, out1_ref, h_ref, ab_ref, w2_ref, b2_ref,
             y_ref):
        p = jnp.concatenate([p_ref[0], p_ref[1]], axis=1)
        dv = dinv_ref[...]
        alpha = ab_ref[:, 0:1]
        beta = ab_ref[:, 1:2]
        o1 = out1_ref[...]
        out2 = alpha * (dv * p + dv * dv * o1) + beta * h_ref[...]
        y = lax.dot_general(out2, w2_ref[...], (((1,), (1,)), ((), ())),
                            precision=lax.Precision.HIGHEST) + b2_ref[...]
        m = jnp.max(y, axis=1, keepdims=True)
        lse = m + jnp.log(jnp.sum(jnp.exp(y - m), axis=1, keepdims=True))
        y_ref[...] = y - lse

    return pl.pallas_call(
        body,
        grid=(g,),
        in_specs=[
            pl.BlockSpec((2, blk, w2), lambda i: (0, i, 0)),
            pl.BlockSpec((blk, 1), lambda i: (i, 0)),
            pl.BlockSpec((blk, dh), lambda i: (i, 0)),
            pl.BlockSpec((blk, dh), lambda i: (i, 0)),
            pl.BlockSpec((blk, 2), lambda i: (i, 0)),
            pl.BlockSpec((dout, dh), lambda i: (0, 0)),
            pl.BlockSpec((1, dout), lambda i: (0, 0)),
        ],
        out_specs=pl.BlockSpec((blk, dout), lambda i: (i, 0)),
        out_shape=jax.ShapeDtypeStruct((n, dout), jnp.float32),
    )(parts, dinv, out1, h, ab, w2_, b2r)


def kernel(x, edge_index, W1, b1, W2, b2):
    n, _ = x.shape
    dh = W1.shape[0]
    e = edge_index.shape[1]
    blk = 1000 if n % 1000 == 0 else 8

    npad = -(-(n + 1) // (NS * CHUNK)) * (NS * CHUNK)
    nchunks = -(-e // CHUNK)
    nchunks = -(-nchunks // (NS * 4)) * (NS * 4)
    epad = (nchunks + 2) * CHUNK - e

    row = edge_index[0].astype(jnp.int16)
    col = edge_index[1].astype(jnp.int16)
    padv = jnp.full((epad,), n, jnp.int16)
    rowc = lax.bitcast_convert_type(
        jnp.concatenate([row, padv]).reshape(nchunks + 2, CHUNK // 2, 2),
        jnp.int32)
    colc = lax.bitcast_convert_type(
        jnp.concatenate([col, padv]).reshape(nchunks + 2, CHUNK // 2, 2),
        jnp.int32)

    degp = _make_deg_pass(npad, nchunks)(rowc)

    h, t1a, t1b, dinv = _tc_prep(x, W1, b1.reshape(1, -1), degp, npad, blk)

    spmm_a = _make_spmm_split(npad, (dh + 32) // 2, nchunks, deep=False)
    p1 = spmm_a(t1a, t1b, colc, rowc)

    out1, t2a, t2b, ab = _tc_combine1(p1, dinv, h, npad, blk)

    spmm_b = _make_spmm_split(npad, dh // 2, nchunks)
    p2 = spmm_b(t2a, t2b, colc, rowc)

    return _tc_combine2(p2, dinv, out1, h, ab, W2, b2.reshape(1, -1), blk)


# speedup vs baseline: 1.4916x; 1.4916x over previous
import functools

import jax
import jax.numpy as jnp
from jax import lax
from jax.experimental import pallas as pl
from jax.experimental.pallas import tpu as pltpu
from jax.experimental.pallas import tpu_sc as plsc

MU_C = 0.1
NC = 2
NS = 16
CHUNK = 128


def _fill_f32(ref, rows, width, value):
    vals = jnp.full((16,), value, jnp.float32)

    def body(i, carry):
        for t in range(width // 16):
            ref[i, pl.ds(t * 16, 16)] = vals
        return carry

    lax.fori_loop(0, rows, body, 0)


def _make_deg_pass(npad, nchunks):
    w = 16
    mesh = plsc.VectorSubcoreMesh(core_axis_name="c", subcore_axis_name="s")
    cpt = nchunks // (NC * NS)
    rps = npad // NS
    ncopy = rps // CHUNK

    scratch = [
        pltpu.VMEM((CHUNK, w), jnp.float32),
        pltpu.VMEM((CHUNK,), jnp.int32),
        pltpu.VMEM_SHARED((npad, w), jnp.float32),
        pltpu.SemaphoreType.DMA,
    ]
    out_type = jax.ShapeDtypeStruct((NC, npad, w), jnp.float32)

    @functools.partial(
        pl.kernel, out_type=out_type, mesh=mesh, scratch_types=scratch,
        compiler_params=pltpu.CompilerParams(use_tc_tiling_on_sc=False))
    def deg_pass(row_hbm, out_hbm, ones_v, ridx_v, acc, sem):
        c = lax.axis_index("c")
        s = lax.axis_index("s")
        tid = c * NS + s

        _fill_f32(ones_v, CHUNK, w, 0.0)
        for k in range(ncopy):
            pltpu.sync_copy(ones_v,
                            acc.at[pl.ds(s * rps + k * CHUNK, CHUNK)])
        _fill_f32(ones_v, CHUNK, w, 1.0)
        plsc.subcore_barrier()

        def body(j, carry):
            ch = tid * cpt + j
            pltpu.sync_copy(row_hbm.at[ch], ridx_v)
            pltpu.sync_copy(ones_v, acc.at[ridx_v], add=True)
            return carry

        lax.fori_loop(0, cpt, body, 0)
        plsc.subcore_barrier()
        pltpu.sync_copy(acc.at[pl.ds(s * rps, rps)],
                        out_hbm.at[c, pl.ds(s * rps, rps)])

    return deg_pass


def _make_spmm_split(npad, w2, nchunks, deep=True):
    mesh = plsc.VectorSubcoreMesh(core_axis_name="c", subcore_axis_name="s")
    cps = nchunks // NS
    rps = npad // NS
    ncopy = rps // CHUNK

    nset = 4 if deep else 2

    scratch = [
        [pltpu.VMEM((CHUNK, w2), jnp.float32) for _ in range(nset)],
        [pltpu.VMEM((CHUNK,), jnp.int32) for _ in range(nset)],
        [pltpu.VMEM((CHUNK,), jnp.int32) for _ in range(nset)],
        pltpu.VMEM_SHARED((npad, w2), jnp.float32),
        pltpu.VMEM_SHARED((npad, w2), jnp.float32),
        [pltpu.SemaphoreType.DMA for _ in range(nset)],
        [pltpu.SemaphoreType.DMA for _ in range(nset)],
        [pltpu.SemaphoreType.DMA for _ in range(nset)],
    ]
    out_type = jax.ShapeDtypeStruct((NC, npad, w2), jnp.float32)

    @functools.partial(
        pl.kernel, out_type=out_type, mesh=mesh, scratch_types=scratch,
        compiler_params=pltpu.CompilerParams(use_tc_tiling_on_sc=False))
    def spmm(t0_hbm, t1_hbm, col_hbm, row_hbm, out_hbm, bufs, rvs, cvs,
             tbl, acc, gsems, isems, ssems):
        c = lax.axis_index("c")
        s = lax.axis_index("s")

        for cc in range(NC):
            src = t0_hbm if cc == 0 else t1_hbm
            for ss in range(NS):
                @pl.when(jnp.logical_and(c == cc, s == ss))
                def _():
                    pltpu.sync_copy(src.at[pl.ds(ss * rps, rps)],
                                    tbl.at[pl.ds(ss * rps, rps)])

        _fill_f32(bufs[0], CHUNK, w2, 0.0)
        for k in range(ncopy):
            pltpu.sync_copy(bufs[0],
                            acc.at[pl.ds(s * rps + k * CHUNK, CHUNK)])
        plsc.subcore_barrier()

        base0 = s * cps

        def fetch_idx(p, ch):
            pltpu.async_copy(col_hbm.at[ch], cvs[p], isems[p])
            pltpu.async_copy(row_hbm.at[ch], rvs[p], isems[p])

        def wait_idx(p, ch):
            pltpu.make_async_copy(col_hbm.at[ch], cvs[p], isems[p]).wait()
            pltpu.make_async_copy(row_hbm.at[ch], rvs[p], isems[p]).wait()

        def wait_scat(p):
            pltpu.make_async_copy(bufs[p], acc.at[rvs[p]], ssems[p]).wait()

        if not deep:
            fetch_idx(0, base0)
            fetch_idx(1, base0 + 1)

            def body2(g, carry):
                ch = base0 + 2 * g
                wait_idx(0, ch)
                wait_idx(1, ch + 1)
                g0 = pltpu.async_copy(tbl.at[cvs[0]], bufs[0], gsems[0])
                g1 = pltpu.async_copy(tbl.at[cvs[1]], bufs[1], gsems[1])
                g0.wait()
                pltpu.sync_copy(bufs[0], acc.at[rvs[0]], add=True)
                fetch_idx(0, ch + 2)
                g1.wait()
                pltpu.sync_copy(bufs[1], acc.at[rvs[1]], add=True)
                fetch_idx(1, ch + 3)
                return carry

            lax.fori_loop(0, cps // 2, body2, 0)
            wait_idx(0, base0)
            wait_idx(1, base0)
            plsc.subcore_barrier()
            pltpu.sync_copy(acc.at[pl.ds(s * rps, rps)],
                            out_hbm.at[c, pl.ds(s * rps, rps)])
            return

        fetch_idx(0, base0)
        fetch_idx(1, base0 + 1)
        wait_idx(0, base0)
        pltpu.async_copy(tbl.at[cvs[0]], bufs[0], gsems[0])
        for p in (2, 3):
            pltpu.async_copy(bufs[p], acc.at[pl.ds(npad - CHUNK, CHUNK)],
                             ssems[p])

        def stage(p, ch):
            q = (p + 1) % nset
            r = (p + 2) % nset
            pltpu.make_async_copy(tbl.at[cvs[p]], bufs[p], gsems[p]).wait()
            pltpu.async_copy(bufs[p], acc.at[rvs[p]], ssems[p], add=True)
            wait_idx(q, ch + 1)
            pltpu.async_copy(tbl.at[cvs[q]], bufs[q], gsems[q])
            wait_scat(r)
            fetch_idx(r, ch + 2)

        def body(g, carry):
            base = base0 + g * nset
            for p in range(nset):
                stage(p, base + p)
            return carry

        lax.fori_loop(0, cps // nset, body, 0)

        pltpu.make_async_copy(tbl.at[cvs[0]], bufs[0], gsems[0]).wait()
        wait_idx(1, base0)
        wait_scat((cps - 2) % nset)
        wait_scat((cps - 1) % nset)
        plsc.subcore_barrier()
        pltpu.sync_copy(acc.at[pl.ds(s * rps, rps)],
                        out_hbm.at[c, pl.ds(s * rps, rps)])

    return spmm


def _tc_prep(x, w1, b1r, degp, npad, blk):
    n, din = x.shape
    dh = w1.shape[0]
    wt = dh + 32
    w2 = wt // 2
    g = n // blk

    def body(x_ref, w1_ref, b1_ref, degp_ref, h_ref, ta_ref, tb_ref,
             dinv_ref):
        xx = x_ref[...]
        h = lax.dot_general(xx, w1_ref[...], (((1,), (1,)), ((), ())),
                            precision=lax.Precision.HIGHEST)
        h = jnp.maximum(h + b1_ref[...], 0.0)
        deg = degp_ref[0, :, 0] + degp_ref[1, :, 0] + 1.0
        dinv = lax.rsqrt(deg)[:, None]
        h_ref[...] = h
        dinv_ref[...] = dinv
        t1 = jnp.concatenate(
            [dinv * h, dinv, jnp.zeros((blk, wt - dh - 1), jnp.float32)],
            axis=1)
        ta_ref[...] = t1[:, :w2]
        tb_ref[...] = t1[:, w2:]

    return pl.pallas_call(
        body,
        grid=(g,),
        in_specs=[
            pl.BlockSpec((blk, din), lambda i: (i, 0)),
            pl.BlockSpec((dh, din), lambda i: (0, 0)),
            pl.BlockSpec((1, dh), lambda i: (0, 0)),
            pl.BlockSpec((2, blk, 16), lambda i: (0, i, 0)),
        ],
        out_specs=[
            pl.BlockSpec((blk, dh), lambda i: (i, 0)),
            pl.BlockSpec((blk, w2), lambda i: (i, 0)),
            pl.BlockSpec((blk, w2), lambda i: (i, 0)),
            pl.BlockSpec((blk, 1), lambda i: (i, 0)),
        ],
        out_shape=[
            jax.ShapeDtypeStruct((n, dh), jnp.float32),
            jax.ShapeDtypeStruct((npad, w2), jnp.float32),
            jax.ShapeDtypeStruct((npad, w2), jnp.float32),
            jax.ShapeDtypeStruct((n, 1), jnp.float32),
        ],
    )(x, w1, b1r, degp)


def _tc_combine1(parts, dinv, h, npad, blk):
    n, dh = h.shape
    w2 = parts.shape[2]
    g = n // blk

    def body(p_ref, dinv_ref, h_ref, out1_ref, ta_ref, tb_ref, ab_ref):
        p = jnp.concatenate([p_ref[0], p_ref[1]], axis=1)
        s128 = p[:, :dh]
        scol = p[:, dh:dh + 1]
        dv = dinv_ref[...]
        deg_m = dv * scol + dv * dv
        alpha = 1.0 / (MU_C + deg_m)
        beta = MU_C * alpha
        hh = h_ref[...]
        out1 = alpha * (dv * s128 + dv * dv * hh) + beta * hh
        out1_ref[...] = out1
        t2 = dv * out1
        ta_ref[...] = t2[:, :dh // 2]
        tb_ref[...] = t2[:, dh // 2:]
        ab_ref[...] = jnp.concatenate([alpha, beta], axis=1)

    return pl.pallas_call(
        body,
        grid=(g,),
        in_specs=[
            pl.BlockSpec((2, blk, w2), lambda i: (0, i, 0)),
            pl.BlockSpec((blk, 1), lambda i: (i, 0)),
            pl.BlockSpec((blk, dh), lambda i: (i, 0)),
        ],
        out_specs=[
            pl.BlockSpec((blk, dh), lambda i: (i, 0)),
            pl.BlockSpec((blk, dh // 2), lambda i: (i, 0)),
            pl.BlockSpec((blk, dh // 2), lambda i: (i, 0)),
            pl.BlockSpec((blk, 2), lambda i: (i, 0)),
        ],
        out_shape=[
            jax.ShapeDtypeStruct((n, dh), jnp.float32),
            jax.ShapeDtypeStruct((npad, dh // 2), jnp.float32),
            jax.ShapeDtypeStruct((npad, dh // 2), jnp.float32),
            jax.ShapeDtypeStruct((n, 2), jnp.float32),
        ],
    )(parts, dinv, h)


def _tc_combine2(parts, dinv, out1, h, ab, w2_, b2r, blk):
    n, dh = h.shape
    dout = w2_.shape[0]
    w2 = parts.shape[2]
    g = n // blk

    def body(p_ref, dinv_ref, out1_ref, h_ref, ab_ref, w2_ref, b2_ref,
             y_ref):
        p = jnp.concatenate([p_ref[0], p_ref[1]], axis=1)
        dv = dinv_ref[...]
        alpha = ab_ref[:, 0:1]
        beta = ab_ref[:, 1:2]
        o1 = out1_ref[...]
        out2 = alpha * (dv * p + dv * dv * o1) + beta * h_ref[...]
        y = lax.dot_general(out2, w2_ref[...], (((1,), (1,)), ((), ())),
                            precision=lax.Precision.HIGHEST) + b2_ref[...]
        m = jnp.max(y, axis=1, keepdims=True)
        lse = m + jnp.log(jnp.sum(jnp.exp(y - m), axis=1, keepdims=True))
        y_ref[...] = y - lse

    return pl.pallas_call(
        body,
        grid=(g,),
        in_specs=[
            pl.BlockSpec((2, blk, w2), lambda i: (0, i, 0)),
            pl.BlockSpec((blk, 1), lambda i: (i, 0)),
            pl.BlockSpec((blk, dh), lambda i: (i, 0)),
            pl.BlockSpec((blk, dh), lambda i: (i, 0)),
            pl.BlockSpec((blk, 2), lambda i: (i, 0)),
            pl.BlockSpec((dout, dh), lambda i: (0, 0)),
            pl.BlockSpec((1, dout), lambda i: (0, 0)),
        ],
        out_specs=pl.BlockSpec((blk, dout), lambda i: (i, 0)),
        out_shape=jax.ShapeDtypeStruct((n, dout), jnp.float32),
    )(parts, dinv, out1, h, ab, w2_, b2r)


def kernel(x, edge_index, W1, b1, W2, b2):
    n, _ = x.shape
    dh = W1.shape[0]
    e = edge_index.shape[1]
    blk = 1000 if n % 1000 == 0 else 8

    npad = -(-(n + 1) // (NS * CHUNK)) * (NS * CHUNK)
    nchunks = -(-e // CHUNK)
    nchunks = -(-nchunks // (NS * 4)) * (NS * 4)
    epad = (nchunks + 2) * CHUNK - e

    row = edge_index[0].astype(jnp.int32)
    col = edge_index[1].astype(jnp.int32)
    padv = jnp.full((epad,), n, jnp.int32)
    rowc = jnp.concatenate([row, padv]).reshape(nchunks + 2, CHUNK)
    colc = jnp.concatenate([col, padv]).reshape(nchunks + 2, CHUNK)

    degp = _make_deg_pass(npad, nchunks)(rowc)

    h, t1a, t1b, dinv = _tc_prep(x, W1, b1.reshape(1, -1), degp, npad, blk)

    spmm_a = _make_spmm_split(npad, (dh + 32) // 2, nchunks, deep=False)
    p1 = spmm_a(t1a, t1b, colc, rowc)

    out1, t2a, t2b, ab = _tc_combine1(p1, dinv, h, npad, blk)

    spmm_b = _make_spmm_split(npad, dh // 2, nchunks)
    p2 = spmm_b(t2a, t2b, colc, rowc)

    return _tc_combine2(p2, dinv, out1, h, ab, W2, b2.reshape(1, -1), blk)


# speedup vs baseline: 1.5477x; 1.0376x over previous
import functools

import jax
import jax.numpy as jnp
from jax import lax
from jax.experimental import pallas as pl
from jax.experimental.pallas import tpu as pltpu
from jax.experimental.pallas import tpu_sc as plsc

MU_C = 0.1
NC = 2
NS = 16
CHUNK = 128


def _fill_f32(ref, rows, width, value):
    vals = jnp.full((16,), value, jnp.float32)

    def body(i, carry):
        for t in range(width // 16):
            ref[i, pl.ds(t * 16, 16)] = vals
        return carry

    lax.fori_loop(0, rows, body, 0)


def _make_deg_pass(npad, nchunks):
    w = 16
    mesh = plsc.VectorSubcoreMesh(core_axis_name="c", subcore_axis_name="s")
    cpt = nchunks // (NC * NS)
    rps = npad // NS
    ncopy = rps // CHUNK

    scratch = [
        pltpu.VMEM((CHUNK, w), jnp.float32),
        [pltpu.VMEM((CHUNK,), jnp.int32) for _ in range(2)],
        pltpu.VMEM_SHARED((npad, w), jnp.float32),
        [pltpu.SemaphoreType.DMA for _ in range(2)],
    ]
    out_type = jax.ShapeDtypeStruct((NC, npad, w), jnp.float32)

    @functools.partial(
        pl.kernel, out_type=out_type, mesh=mesh, scratch_types=scratch,
        compiler_params=pltpu.CompilerParams(use_tc_tiling_on_sc=False))
    def deg_pass(row_hbm, out_hbm, ones_v, ridx, acc, isems):
        c = lax.axis_index("c")
        s = lax.axis_index("s")
        tid = c * NS + s

        def fetch(p, ch):
            pltpu.async_copy(row_hbm.at[ch], ridx[p], isems[p])

        def wait_fetch(p, ch):
            pltpu.make_async_copy(row_hbm.at[ch], ridx[p], isems[p]).wait()

        base0 = tid * cpt
        fetch(0, base0)
        fetch(1, base0 + 1)
        _fill_f32(ones_v, CHUNK, w, 0.0)
        for k in range(ncopy):
            pltpu.sync_copy(ones_v,
                            acc.at[pl.ds(s * rps + k * CHUNK, CHUNK)])
        _fill_f32(ones_v, CHUNK, w, 1.0)
        plsc.subcore_barrier()

        def body(j, carry):
            ch = base0 + 2 * j
            for p in range(2):
                wait_fetch(p, ch + p)
                pltpu.sync_copy(ones_v, acc.at[ridx[p]], add=True)
                fetch(p, ch + p + 2)
            return carry

        lax.fori_loop(0, cpt // 2, body, 0)
        wait_fetch(0, base0)
        wait_fetch(1, base0)
        plsc.subcore_barrier()
        pltpu.sync_copy(acc.at[pl.ds(s * rps, rps)],
                        out_hbm.at[c, pl.ds(s * rps, rps)])

    return deg_pass


def _make_spmm_split(npad, w2, nchunks, deep=True):
    mesh = plsc.VectorSubcoreMesh(core_axis_name="c", subcore_axis_name="s")
    cps = nchunks // NS
    rps = npad // NS
    ncopy = rps // CHUNK

    nset = 4 if deep else 2

    scratch = [
        [pltpu.VMEM((CHUNK, w2), jnp.float32) for _ in range(nset)],
        [pltpu.VMEM((CHUNK,), jnp.int32) for _ in range(nset)],
        [pltpu.VMEM((CHUNK,), jnp.int32) for _ in range(nset)],
        pltpu.VMEM_SHARED((npad, w2), jnp.float32),
        pltpu.VMEM_SHARED((npad, w2), jnp.float32),
        [pltpu.SemaphoreType.DMA for _ in range(nset)],
        [pltpu.SemaphoreType.DMA for _ in range(nset)],
        [pltpu.SemaphoreType.DMA for _ in range(nset)],
    ]
    out_type = jax.ShapeDtypeStruct((NC, npad, w2), jnp.float32)

    @functools.partial(
        pl.kernel, out_type=out_type, mesh=mesh, scratch_types=scratch,
        compiler_params=pltpu.CompilerParams(use_tc_tiling_on_sc=False))
    def spmm(t0_hbm, t1_hbm, col_hbm, row_hbm, out_hbm, bufs, rvs, cvs,
             tbl, acc, gsems, isems, ssems):
        c = lax.axis_index("c")
        s = lax.axis_index("s")

        for cc in range(NC):
            src = t0_hbm if cc == 0 else t1_hbm
            for ss in range(NS):
                @pl.when(jnp.logical_and(c == cc, s == ss))
                def _():
                    pltpu.sync_copy(src.at[pl.ds(ss * rps, rps)],
                                    tbl.at[pl.ds(ss * rps, rps)])

        _fill_f32(bufs[0], CHUNK, w2, 0.0)
        for k in range(ncopy):
            pltpu.sync_copy(bufs[0],
                            acc.at[pl.ds(s * rps + k * CHUNK, CHUNK)])
        plsc.subcore_barrier()

        base0 = s * cps

        def fetch_idx(p, ch):
            pltpu.async_copy(col_hbm.at[ch], cvs[p], isems[p])
            pltpu.async_copy(row_hbm.at[ch], rvs[p], isems[p])

        def wait_idx(p, ch):
            pltpu.make_async_copy(col_hbm.at[ch], cvs[p], isems[p]).wait()
            pltpu.make_async_copy(row_hbm.at[ch], rvs[p], isems[p]).wait()

        def wait_scat(p):
            pltpu.make_async_copy(bufs[p], acc.at[rvs[p]], ssems[p]).wait()

        if not deep:
            fetch_idx(0, base0)
            fetch_idx(1, base0 + 1)

            def body2(g, carry):
                ch = base0 + 2 * g
                wait_idx(0, ch)
                wait_idx(1, ch + 1)
                g0 = pltpu.async_copy(tbl.at[cvs[0]], bufs[0], gsems[0])
                g1 = pltpu.async_copy(tbl.at[cvs[1]], bufs[1], gsems[1])
                g0.wait()
                pltpu.sync_copy(bufs[0], acc.at[rvs[0]], add=True)
                fetch_idx(0, ch + 2)
                g1.wait()
                pltpu.sync_copy(bufs[1], acc.at[rvs[1]], add=True)
                fetch_idx(1, ch + 3)
                return carry

            lax.fori_loop(0, cps // 2, body2, 0)
            wait_idx(0, base0)
            wait_idx(1, base0)
            plsc.subcore_barrier()
            pltpu.sync_copy(acc.at[pl.ds(s * rps, rps)],
                            out_hbm.at[c, pl.ds(s * rps, rps)])
            return

        fetch_idx(0, base0)
        fetch_idx(1, base0 + 1)
        wait_idx(0, base0)
        pltpu.async_copy(tbl.at[cvs[0]], bufs[0], gsems[0])
        for p in (2, 3):
            pltpu.async_copy(bufs[p], acc.at[pl.ds(npad - CHUNK, CHUNK)],
                             ssems[p])

        def stage(p, ch):
            q = (p + 1) % nset
            r = (p + 2) % nset
            pltpu.make_async_copy(tbl.at[cvs[p]], bufs[p], gsems[p]).wait()
            pltpu.async_copy(bufs[p], acc.at[rvs[p]], ssems[p], add=True)
            wait_idx(q, ch + 1)
            pltpu.async_copy(tbl.at[cvs[q]], bufs[q], gsems[q])
            wait_scat(r)
            fetch_idx(r, ch + 2)

        def body(g, carry):
            base = base0 + g * nset
            for p in range(nset):
                stage(p, base + p)
            return carry

        lax.fori_loop(0, cps // nset, body, 0)

        pltpu.make_async_copy(tbl.at[cvs[0]], bufs[0], gsems[0]).wait()
        wait_idx(1, base0)
        wait_scat((cps - 2) % nset)
        wait_scat((cps - 1) % nset)
        plsc.subcore_barrier()
        pltpu.sync_copy(acc.at[pl.ds(s * rps, rps)],
                        out_hbm.at[c, pl.ds(s * rps, rps)])

    return spmm


def _tc_prep(x, w1, b1r, degp, npad, blk):
    n, din = x.shape
    dh = w1.shape[0]
    wt = dh + 32
    w2 = wt // 2
    g = n // blk

    def body(x_ref, w1_ref, b1_ref, degp_ref, h_ref, ta_ref, tb_ref,
             dinv_ref):
        xx = x_ref[...]
        h = lax.dot_general(xx, w1_ref[...], (((1,), (1,)), ((), ())),
                            precision=lax.Precision.HIGHEST)
        h = jnp.maximum(h + b1_ref[...], 0.0)
        deg = degp_ref[0, :, 0] + degp_ref[1, :, 0] + 1.0
        dinv = lax.rsqrt(deg)[:, None]
        h_ref[...] = h
        dinv_ref[...] = dinv
        t1 = jnp.concatenate(
            [dinv * h, dinv, jnp.zeros((blk, wt - dh - 1), jnp.float32)],
            axis=1)
        ta_ref[...] = t1[:, :w2]
        tb_ref[...] = t1[:, w2:]

    return pl.pallas_call(
        body,
        grid=(g,),
        in_specs=[
            pl.BlockSpec((blk, din), lambda i: (i, 0)),
            pl.BlockSpec((dh, din), lambda i: (0, 0)),
            pl.BlockSpec((1, dh), lambda i: (0, 0)),
            pl.BlockSpec((2, blk, 16), lambda i: (0, i, 0)),
        ],
        out_specs=[
            pl.BlockSpec((blk, dh), lambda i: (i, 0)),
            pl.BlockSpec((blk, w2), lambda i: (i, 0)),
            pl.BlockSpec((blk, w2), lambda i: (i, 0)),
            pl.BlockSpec((blk, 1), lambda i: (i, 0)),
        ],
        out_shape=[
            jax.ShapeDtypeStruct((n, dh), jnp.float32),
            jax.ShapeDtypeStruct((npad, w2), jnp.float32),
            jax.ShapeDtypeStruct((npad, w2), jnp.float32),
            jax.ShapeDtypeStruct((n, 1), jnp.float32),
        ],
    )(x, w1, b1r, degp)


def _tc_combine1(parts, dinv, h, npad, blk):
    n, dh = h.shape
    w2 = parts.shape[2]
    g = n // blk

    def body(p_ref, dinv_ref, h_ref, out1_ref, ta_ref, tb_ref, ab_ref):
        p = jnp.concatenate([p_ref[0], p_ref[1]], axis=1)
        s128 = p[:, :dh]
        scol = p[:, dh:dh + 1]
        dv = dinv_ref[...]
        deg_m = dv * scol + dv * dv
        alpha = 1.0 / (MU_C + deg_m)
        beta = MU_C * alpha
        hh = h_ref[...]
        out1 = alpha * (dv * s128 + dv * dv * hh) + beta * hh
        out1_ref[...] = out1
        t2 = dv * out1
        ta_ref[...] = t2[:, :dh // 2]
        tb_ref[...] = t2[:, dh // 2:]
        ab_ref[...] = jnp.concatenate([alpha, beta], axis=1)

    return pl.pallas_call(
        body,
        grid=(g,),
        in_specs=[
            pl.BlockSpec((2, blk, w2), lambda i: (0, i, 0)),
            pl.BlockSpec((blk, 1), lambda i: (i, 0)),
            pl.BlockSpec((blk, dh), lambda i: (i, 0)),
        ],
        out_specs=[
            pl.BlockSpec((blk, dh), lambda i: (i, 0)),
            pl.BlockSpec((blk, dh // 2), lambda i: (i, 0)),
            pl.BlockSpec((blk, dh // 2), lambda i: (i, 0)),
            pl.BlockSpec((blk, 2), lambda i: (i, 0)),
        ],
        out_shape=[
            jax.ShapeDtypeStruct((n, dh), jnp.float32),
            jax.ShapeDtypeStruct((npad, dh // 2), jnp.float32),
            jax.ShapeDtypeStruct((npad, dh // 2), jnp.float32),
            jax.ShapeDtypeStruct((n, 2), jnp.float32),
        ],
    )(parts, dinv, h)


def _tc_combine2(parts, dinv, out1, h, ab, w2_, b2r, blk):
    n, dh = h.shape
    dout = w2_.shape[0]
    w2 = parts.shape[2]
    g = n // blk

    def body(p_ref, dinv_ref, out1_ref, h_ref, ab_ref, w2_ref, b2_ref,
             y_ref):
        p = jnp.concatenate([p_ref[0], p_ref[1]], axis=1)
        dv = dinv_ref[...]
        alpha = ab_ref[:, 0:1]
        beta = ab_ref[:, 1:2]
        o1 = out1_ref[...]
        out2 = alpha * (dv * p + dv * dv * o1) + beta * h_ref[...]
        y = lax.dot_general(out2, w2_ref[...], (((1,), (1,)), ((), ())),
                            precision=lax.Precision.HIGHEST) + b2_ref[...]
        m = jnp.max(y, axis=1, keepdims=True)
        lse = m + jnp.log(jnp.sum(jnp.exp(y - m), axis=1, keepdims=True))
        y_ref[...] = y - lse

    return pl.pallas_call(
        body,
        grid=(g,),
        in_specs=[
            pl.BlockSpec((2, blk, w2), lambda i: (0, i, 0)),
            pl.BlockSpec((blk, 1), lambda i: (i, 0)),
            pl.BlockSpec((blk, dh), lambda i: (i, 0)),
            pl.BlockSpec((blk, dh), lambda i: (i, 0)),
            pl.BlockSpec((blk, 2), lambda i: (i, 0)),
            pl.BlockSpec((dout, dh), lambda i: (0, 0)),
            pl.BlockSpec((1, dout), lambda i: (0, 0)),
        ],
        out_specs=pl.BlockSpec((blk, dout), lambda i: (i, 0)),
        out_shape=jax.ShapeDtypeStruct((n, dout), jnp.float32),
    )(parts, dinv, out1, h, ab, w2_, b2r)


def kernel(x, edge_index, W1, b1, W2, b2):
    n, _ = x.shape
    dh = W1.shape[0]
    e = edge_index.shape[1]
    blk = 1000 if n % 1000 == 0 else 8

    npad = -(-(n + 1) // (NS * CHUNK)) * (NS * CHUNK)
    nchunks = -(-e // CHUNK)
    nchunks = -(-nchunks // (NS * 4)) * (NS * 4)
    epad = (nchunks + 2) * CHUNK - e

    row = edge_index[0].astype(jnp.int32)
    col = edge_index[1].astype(jnp.int32)
    padv = jnp.full((epad,), n, jnp.int32)
    rowc = jnp.concatenate([row, padv]).reshape(nchunks + 2, CHUNK)
    colc = jnp.concatenate([col, padv]).reshape(nchunks + 2, CHUNK)

    degp = _make_deg_pass(npad, nchunks)(rowc)

    h, t1a, t1b, dinv = _tc_prep(x, W1, b1.reshape(1, -1), degp, npad, blk)

    spmm_a = _make_spmm_split(npad, (dh + 32) // 2, nchunks, deep=False)
    p1 = spmm_a(t1a, t1b, colc, rowc)

    out1, t2a, t2b, ab = _tc_combine1(p1, dinv, h, npad, blk)

    spmm_b = _make_spmm_split(npad, dh // 2, nchunks)
    p2 = spmm_b(t2a, t2b, colc, rowc)

    return _tc_combine2(p2, dinv, out1, h, ab, W2, b2.reshape(1, -1), blk)
